# baseline (speedup 1.0000x reference)
import functools

import jax
import jax.numpy as jnp
from jax import lax
from jax.experimental import pallas as pl
from jax.experimental.pallas import tpu as pltpu
from jax.experimental.pallas import tpu_sc as plsc

_CHUNK_IN = 32768
_CHUNK_OUT = _CHUNK_IN // 4
_NVREG = _CHUNK_OUT // 16


def _sc_body(words_per_w, nchunk, x_hbm, out_hbm,
             in0, in1, ou0, ou1, si0, si1, so0, so1):
    cid = lax.axis_index("c")
    sid = lax.axis_index("s")
    wid = sid * 2 + cid
    slab = wid // 4
    quarter = wid % 4
    bidx = slab // 2
    ct = slab % 2
    in_base = quarter * words_per_w

    def in_copy(g, buf, sem):
        ib = pl.multiple_of(in_base + g * _CHUNK_IN, _CHUNK_IN)
        return pltpu.make_async_copy(
            x_hbm.at[bidx, ct, pl.ds(ib, _CHUNK_IN)], buf, sem)

    def out_copy(g, buf, sem):
        ob = pl.multiple_of((in_base + g * _CHUNK_IN) // 4, _CHUNK_OUT)
        return pltpu.make_async_copy(
            buf, out_hbm.at[bidx, ct, pl.ds(ob, _CHUNK_OUT)], sem)

    in_copy(0, in0, si0).start()
    in_copy(1, in1, si1).start()

    bufs = ((in0, ou0, si0, so0), (in1, ou1, si1, so1))
    iota4 = jnp.arange(16, dtype=jnp.int32) * 4

    def pair(p, carry):
        for b in range(2):
            ibuf, obuf, isem, osem = bufs[b]
            g = 2 * p + b
            in_copy(g, ibuf, isem).wait()

            @pl.when(p > 0)
            def _():
                out_copy(g, obuf, osem).wait()

            @plsc.parallel_loop(0, _NVREG, step=1, unroll=4)
            def _(v):
                pt = lax.shift_right_logical(v, 6)
                c = lax.bitwise_and(lax.shift_right_logical(v, 3), 7)
                lo = lax.bitwise_and(v, 7)
                base0 = (lax.shift_left(pt, 12)
                         + lax.shift_left(lax.shift_right_logical(lo, 1), 10)
                         + lax.shift_left(c, 7)
                         + lax.shift_left(lax.bitwise_and(lo, 1), 6))
                cols = iota4 + base0
                acc = (plsc.load_gather(ibuf, [cols])
                       + plsc.load_gather(ibuf, [cols + 1])) + (
                      plsc.load_gather(ibuf, [cols + 2])
                       + plsc.load_gather(ibuf, [cols + 3]))
                obuf[pl.ds(v * 16, 16)] = acc * 0.25

            out_copy(g, obuf, osem).start()

            @pl.when(g + 2 < nchunk)
            def _():
                in_copy(g + 2, ibuf, isem).start()
        return carry

    lax.fori_loop(0, nchunk // 2, pair, 0)
    out_copy(nchunk - 2, ou0, so0).wait()
    out_copy(nchunk - 1, ou1, so1).wait()


def kernel(x, groups):
    batch, npix_fine, channels = x.shape
    npix_coarse, n_children = groups.shape
    assert channels == 16 and n_children == 4
    assert npix_fine % 128 == 0 and npix_coarse % 128 == 0

    info = plsc.get_sparse_core_info()
    nw = info.num_cores * info.num_subcores
    nslab = batch * (channels // 8)
    wps = nw // nslab
    slab_words = npix_fine * 8
    words_per_w = slab_words // wps
    nchunk = words_per_w // _CHUNK_IN
    assert nchunk * _CHUNK_IN == words_per_w and nchunk % 2 == 0

    xv = (x.reshape(batch, npix_fine // 128, 128, 2, 8)
          .transpose(0, 3, 1, 4, 2)
          .reshape(batch, 2, slab_words))

    mesh = plsc.VectorSubcoreMesh(core_axis_name="c", subcore_axis_name="s")
    body = functools.partial(_sc_body, words_per_w, nchunk)
    out = pl.kernel(
        body,
        out_type=jax.ShapeDtypeStruct((batch, 2, slab_words // 4),
                                      jnp.float32),
        mesh=mesh,
        scratch_types=[
            pltpu.VMEM((_CHUNK_IN,), jnp.float32),
            pltpu.VMEM((_CHUNK_IN,), jnp.float32),
            pltpu.VMEM((_CHUNK_OUT,), jnp.float32),
            pltpu.VMEM((_CHUNK_OUT,), jnp.float32),
            pltpu.SemaphoreType.DMA,
            pltpu.SemaphoreType.DMA,
            pltpu.SemaphoreType.DMA,
            pltpu.SemaphoreType.DMA,
        ],
        compiler_params=pltpu.CompilerParams(use_tc_tiling_on_sc=False,
                                             needs_layout_passes=False),
    )(xv)
    return (out.reshape(batch, 2, npix_coarse // 128, 8, 128)
            .transpose(0, 2, 4, 1, 3)
            .reshape(batch, npix_coarse, channels))


# speedup vs baseline: 1.0283x; 1.0283x over previous
import functools

import jax
import jax.numpy as jnp
from jax import lax
from jax.experimental import pallas as pl
from jax.experimental.pallas import tpu as pltpu
from jax.experimental.pallas import tpu_sc as plsc

_CHUNK_IN = 49152
_CHUNK_OUT = _CHUNK_IN // 4
_NVREG = _CHUNK_OUT // 16


def _sc_body(words_per_w, nchunk, x_hbm, out_hbm,
             in0, in1, ou0, ou1, si0, si1, so0, so1):
    cid = lax.axis_index("c")
    sid = lax.axis_index("s")
    wid = sid * 2 + cid
    slab = wid // 4
    quarter = wid % 4
    bidx = slab // 2
    ct = slab % 2
    in_base = quarter * words_per_w

    def in_copy(g, buf, sem):
        ib = pl.multiple_of(in_base + g * _CHUNK_IN, _CHUNK_IN)
        return pltpu.make_async_copy(
            x_hbm.at[bidx, ct, pl.ds(ib, _CHUNK_IN)], buf, sem)

    def out_copy(g, buf, sem):
        ob = pl.multiple_of((in_base + g * _CHUNK_IN) // 4, _CHUNK_OUT)
        return pltpu.make_async_copy(
            buf, out_hbm.at[bidx, ct, pl.ds(ob, _CHUNK_OUT)], sem)

    in_copy(0, in0, si0).start()
    in_copy(1, in1, si1).start()

    bufs = ((in0, ou0, si0, so0), (in1, ou1, si1, so1))
    iota4 = jnp.arange(16, dtype=jnp.int32) * 4

    def pair(p, carry):
        for b in range(2):
            ibuf, obuf, isem, osem = bufs[b]
            g = 2 * p + b
            in_copy(g, ibuf, isem).wait()

            @pl.when(p > 0)
            def _():
                out_copy(g, obuf, osem).wait()

            @plsc.parallel_loop(0, _NVREG, step=1, unroll=4)
            def _(v):
                pt = lax.shift_right_logical(v, 6)
                c = lax.bitwise_and(lax.shift_right_logical(v, 3), 7)
                lo = lax.bitwise_and(v, 7)
                base0 = (lax.shift_left(pt, 12)
                         + lax.shift_left(lax.shift_right_logical(lo, 1), 10)
                         + lax.shift_left(c, 7)
                         + lax.shift_left(lax.bitwise_and(lo, 1), 6))
                cols = iota4 + base0
                acc = (plsc.load_gather(ibuf, [cols])
                       + plsc.load_gather(ibuf, [cols + 1])) + (
                      plsc.load_gather(ibuf, [cols + 2])
                       + plsc.load_gather(ibuf, [cols + 3]))
                obuf[pl.ds(v * 16, 16)] = acc * 0.25

            out_copy(g, obuf, osem).start()

            @pl.when(g + 2 < nchunk)
            def _():
                in_copy(g + 2, ibuf, isem).start()
        return carry

    lax.fori_loop(0, nchunk // 2, pair, 0)
    out_copy(nchunk - 2, ou0, so0).wait()
    out_copy(nchunk - 1, ou1, so1).wait()


def kernel(x, groups):
    batch, npix_fine, channels = x.shape
    npix_coarse, n_children = groups.shape
    assert channels == 16 and n_children == 4
    assert npix_fine % 128 == 0 and npix_coarse % 128 == 0

    info = plsc.get_sparse_core_info()
    nw = info.num_cores * info.num_subcores
    nslab = batch * (channels // 8)
    wps = nw // nslab
    slab_words = npix_fine * 8
    words_per_w = slab_words // wps
    nchunk = words_per_w // _CHUNK_IN
    assert nchunk * _CHUNK_IN == words_per_w and nchunk % 2 == 0

    xv = (x.reshape(batch, npix_fine // 128, 128, 2, 8)
          .transpose(0, 3, 1, 4, 2)
          .reshape(batch, 2, slab_words))

    mesh = plsc.VectorSubcoreMesh(core_axis_name="c", subcore_axis_name="s")
    body = functools.partial(_sc_body, words_per_w, nchunk)
    out = pl.kernel(
        body,
        out_type=jax.ShapeDtypeStruct((batch, 2, slab_words // 4),
                                      jnp.float32),
        mesh=mesh,
        scratch_types=[
            pltpu.VMEM((_CHUNK_IN,), jnp.float32),
            pltpu.VMEM((_CHUNK_IN,), jnp.float32),
            pltpu.VMEM((_CHUNK_OUT,), jnp.float32),
            pltpu.VMEM((_CHUNK_OUT,), jnp.float32),
            pltpu.SemaphoreType.DMA,
            pltpu.SemaphoreType.DMA,
            pltpu.SemaphoreType.DMA,
            pltpu.SemaphoreType.DMA,
        ],
        compiler_params=pltpu.CompilerParams(use_tc_tiling_on_sc=False,
                                             needs_layout_passes=False),
    )(xv)
    return (out.reshape(batch, 2, npix_coarse // 128, 8, 128)
            .transpose(0, 2, 4, 1, 3)
            .reshape(batch, npix_coarse, channels))
